# baseline (speedup 1.0000x reference)
import functools

import jax
import jax.numpy as jnp
from jax import lax
from jax.experimental import pallas as pl
from jax.experimental.pallas import tpu as pltpu
from jax.experimental.pallas import tpu_sc as plsc

_EPS = 1e-12
_D = 64


def _make_sc_gather(B, b_per_w, num_cores, n_sets):
    mesh = plsc.VectorSubcoreMesh(core_axis_name="c", subcore_axis_name="s")

    @functools.partial(
        pl.kernel,
        mesh=mesh,
        out_type=[jax.ShapeDtypeStruct((B, _D), jnp.float32)] * n_sets,
        scratch_types=[
            pltpu.VMEM((b_per_w,), jnp.int32),
            pltpu.VMEM((b_per_w, _D), jnp.float32),
            pltpu.SemaphoreType.DMA,
        ],
    )
    def sc_gather(table, *rest):
        idx_hbms = rest[:n_sets]
        outs = rest[n_sets:2 * n_sets]
        idx_v, out_v, sem = rest[2 * n_sets:]
        wid = lax.axis_index("s") * num_cores + lax.axis_index("c")
        base = wid * b_per_w

        for idx_hbm, out in zip(idx_hbms, outs):
            pltpu.sync_copy(idx_hbm.at[pl.ds(base, b_per_w)], idx_v)

            def fire_group(g, _):
                o = g * 16
                iv = idx_v[pl.ds(o, 16)]
                for j in range(16):
                    pltpu.async_copy(
                        table.at[pl.ds(iv[j], 1)],
                        out_v.at[pl.ds(o + j, 1)], sem)
                return _

            lax.fori_loop(0, b_per_w // 16, fire_group, None)
            pltpu.make_async_copy(
                out.at[pl.ds(base, b_per_w)], out_v, sem).wait()
            pltpu.sync_copy(out_v, out.at[pl.ds(base, b_per_w)])

    return sc_gather


def _tc_body(ph_ref, pt_ref, nh_ref, nt_ref, pr_ref, nr_ref, wt_ref, b_ref,
             o_ph, o_pe, o_pt, o_nh, o_ne, o_nt):
    def nrm(x):
        s = jnp.sum(x * x, axis=1, keepdims=True)
        return x * lax.rsqrt(jnp.maximum(s, _EPS * _EPS))

    ph = nrm(ph_ref[...])
    pt = nrm(pt_ref[...])
    nh = nrm(nh_ref[...])
    nt = nrm(nt_ref[...])
    pr = nrm(pr_ref[...])
    nr = nrm(nr_ref[...])
    wt = wt_ref[...]
    bb = b_ref[...]

    def edge(h, t, r):
        cat = nrm(jnp.concatenate([h, t], axis=1))
        z = jnp.tanh(jnp.dot(cat, wt, preferred_element_type=jnp.float32) + bb)
        z = nrm(z)
        d = jnp.sum(r * z, axis=1, keepdims=True)
        return nrm(r - d * z)

    o_ph[...] = ph.T
    o_pt[...] = pt.T
    o_nh[...] = nh.T
    o_nt[...] = nt.T
    o_pe[...] = edge(ph, pt, pr).T
    o_ne[...] = edge(nh, nt, nr).T


def _tc_transform(ph_r, pt_r, nh_r, nt_r, pr_r, nr_r, Wt, b2d, block_b):
    B, D = ph_r.shape
    grid = (B // block_b,)
    row_spec = pl.BlockSpec((block_b, D), lambda i: (i, 0))
    col_spec = pl.BlockSpec((D, block_b), lambda i: (0, i))
    outs = pl.pallas_call(
        _tc_body,
        grid=grid,
        in_specs=[row_spec] * 6 + [
            pl.BlockSpec(Wt.shape, lambda i: (0, 0)),
            pl.BlockSpec(b2d.shape, lambda i: (0, 0)),
        ],
        out_specs=[col_spec] * 6,
        out_shape=[jax.ShapeDtypeStruct((D, B), jnp.float32)] * 6,
    )(ph_r, pt_r, nh_r, nt_r, pr_r, nr_r, Wt, b2d)
    return tuple(jnp.transpose(o) for o in outs)


def kernel(ent_embed, rel_embed, W, b, phs, prs, pts, nhs, nrs, nts):
    B = phs.shape[0]
    D = ent_embed.shape[1]
    info = plsc.get_sparse_core_info()
    nw = info.num_cores * info.num_subcores
    b_per_w = B // nw

    gat_rel = _make_sc_gather(B, b_per_w, info.num_cores, 2)
    pr_r, nr_r = gat_rel(rel_embed, prs, nrs)
    gat_ent = _make_sc_gather(B, b_per_w, info.num_cores, 4)
    ph_r, pt_r, nh_r, nt_r = gat_ent(ent_embed, phs, pts, nhs, nts)

    Wt = W.T
    b2d = b.reshape(1, D)
    return _tc_transform(
        ph_r, pt_r, nh_r, nt_r, pr_r, nr_r, Wt, b2d, block_b=2048)


# speedup vs baseline: 1.0154x; 1.0154x over previous
import functools

import jax
import jax.numpy as jnp
from jax import lax
from jax.experimental import pallas as pl
from jax.experimental.pallas import tpu as pltpu
from jax.experimental.pallas import tpu_sc as plsc

_EPS = 1e-12
_D = 64


def _make_sc_gather(B, b_per_w, num_cores):
    mesh = plsc.VectorSubcoreMesh(core_axis_name="c", subcore_axis_name="s")

    @functools.partial(
        pl.kernel,
        mesh=mesh,
        out_type=[jax.ShapeDtypeStruct((B, _D), jnp.float32)] * 6,
        scratch_types=[
            pltpu.VMEM((b_per_w,), jnp.int32),
            pltpu.VMEM((b_per_w // 2, _D), jnp.float32),
            pltpu.VMEM((b_per_w // 2, _D), jnp.float32),
            pltpu.SemaphoreType.DMA,
            pltpu.SemaphoreType.DMA,
        ],
    )
    def sc_gather(ent_hbm, rel_hbm, phs, pts, nhs, nts, prs, nrs,
                  o_ph, o_pt, o_nh, o_nt, o_pr, o_nr,
                  idx_v, buf0, buf1, sem0, sem1):
        wid = lax.axis_index("s") * num_cores + lax.axis_index("c")
        base = wid * b_per_w
        half = b_per_w // 2
        bufs = (buf0, buf1)
        sems = (sem0, sem1)
        sets = (
            (phs, ent_hbm, o_ph), (pts, ent_hbm, o_pt),
            (nhs, ent_hbm, o_nh), (nts, ent_hbm, o_nt),
            (prs, rel_hbm, o_pr), (nrs, rel_hbm, o_nr),
        )
        units = [(si, h) for si in range(6) for h in range(2)]

        def drain(ui):
            si, h = units[ui]
            out = sets[si][2]
            lo = base + h * half
            pltpu.make_async_copy(
                out.at[pl.ds(lo, half)], bufs[ui % 2], sems[ui % 2]).wait()
            pltpu.sync_copy(bufs[ui % 2], out.at[pl.ds(lo, half)])

        for ui, (si, h) in enumerate(units):
            idx_hbm, table, out = sets[si]
            if h == 0:
                pltpu.sync_copy(idx_hbm.at[pl.ds(base, b_per_w)], idx_v)
            buf, sem = bufs[ui % 2], sems[ui % 2]

            def fire_group(g, _, table=table, buf=buf, sem=sem, h=h):
                o = h * half + g * 16
                iv = idx_v[pl.ds(o, 16)]
                for j in range(16):
                    pltpu.async_copy(
                        table.at[pl.ds(iv[j], 1)],
                        buf.at[pl.ds(g * 16 + j, 1)], sem)
                return _

            lax.fori_loop(0, half // 16, fire_group, None)
            if ui >= 1:
                drain(ui - 1)
        drain(len(units) - 1)

    return sc_gather


def _tc_body(ph_ref, pt_ref, nh_ref, nt_ref, pr_ref, nr_ref, wt_ref, b_ref,
             o_ph, o_pe, o_pt, o_nh, o_ne, o_nt):
    def nrm(x):
        s = jnp.sum(x * x, axis=1, keepdims=True)
        return x * lax.rsqrt(jnp.maximum(s, _EPS * _EPS))

    ph = nrm(ph_ref[...])
    pt = nrm(pt_ref[...])
    nh = nrm(nh_ref[...])
    nt = nrm(nt_ref[...])
    pr = nrm(pr_ref[...])
    nr = nrm(nr_ref[...])
    wt = wt_ref[...]
    bb = b_ref[...]

    def edge(h, t, r):
        cat = nrm(jnp.concatenate([h, t], axis=1))
        z = jnp.tanh(jnp.dot(cat, wt, preferred_element_type=jnp.float32) + bb)
        z = nrm(z)
        d = jnp.sum(r * z, axis=1, keepdims=True)
        return nrm(r - d * z)

    o_ph[...] = ph.T
    o_pt[...] = pt.T
    o_nh[...] = nh.T
    o_nt[...] = nt.T
    o_pe[...] = edge(ph, pt, pr).T
    o_ne[...] = edge(nh, nt, nr).T


def _tc_transform(ph_r, pt_r, nh_r, nt_r, pr_r, nr_r, Wt, b2d, block_b):
    B, D = ph_r.shape
    grid = (B // block_b,)
    row_spec = pl.BlockSpec((block_b, D), lambda i: (i, 0))
    col_spec = pl.BlockSpec((D, block_b), lambda i: (0, i))
    outs = pl.pallas_call(
        _tc_body,
        grid=grid,
        in_specs=[row_spec] * 6 + [
            pl.BlockSpec(Wt.shape, lambda i: (0, 0)),
            pl.BlockSpec(b2d.shape, lambda i: (0, 0)),
        ],
        out_specs=[col_spec] * 6,
        out_shape=[jax.ShapeDtypeStruct((D, B), jnp.float32)] * 6,
    )(ph_r, pt_r, nh_r, nt_r, pr_r, nr_r, Wt, b2d)
    return tuple(jnp.transpose(o) for o in outs)


def kernel(ent_embed, rel_embed, W, b, phs, prs, pts, nhs, nrs, nts):
    B = phs.shape[0]
    D = ent_embed.shape[1]
    info = plsc.get_sparse_core_info()
    nw = info.num_cores * info.num_subcores
    b_per_w = B // nw

    gat = _make_sc_gather(B, b_per_w, info.num_cores)
    ph_r, pt_r, nh_r, nt_r, pr_r, nr_r = gat(
        ent_embed, rel_embed, phs, pts, nhs, nts, prs, nrs)

    Wt = W.T
    b2d = b.reshape(1, D)
    return _tc_transform(
        ph_r, pt_r, nh_r, nt_r, pr_r, nr_r, Wt, b2d, block_b=2048)
